# baseline (speedup 1.0000x reference)
import functools

import jax
import jax.numpy as jnp
from jax import lax
from jax.experimental import pallas as pl
from jax.experimental.pallas import tpu as pltpu
from jax.experimental.pallas import tpu_sc as plsc

_ETA = 10
_K = 32
_NC = 2
_NS = 16
_NW = _NC * _NS
_L = 16

_N_ENT = 1000000
_ALIGNED = 999936
_TPG = 7813
_GSTRIDE = _TPG * 1024
_NTILES = 8 * _TPG
_QT = 7812 // 4
_FIRE = 31


@functools.lru_cache(maxsize=None)
def _build_detile():
    mesh = plsc.VectorSubcoreMesh(core_axis_name="c", subcore_axis_name="s")

    @functools.partial(
        pl.kernel,
        out_type=jax.ShapeDtypeStruct((_NTILES, 8, 128), jnp.float32),
        mesh=mesh,
        compiler_params=pltpu.CompilerParams(
            needs_layout_passes=False, use_tc_tiling_on_sc=True),
        scratch_types=[
            pltpu.SemaphoreType.DMA,
        ],
    )
    def detile(entT_hbm, relT_hbm, tail_hbm, out_hbm, sem):
        cid = lax.axis_index("c")
        sid = lax.axis_index("s")
        wid = sid * _NC + cid
        rg = wid % 16
        kr = rg // 4
        q = rg % 4
        row0 = pl.multiple_of(8 * kr, 8)
        t0 = q * _QT

        def run(tab_hbm, tbase):
            def chunk(j, carry):
                base = t0 + j * _FIRE
                cps = []
                for u in range(_FIRE):
                    tix = base + u
                    col = pl.multiple_of(tix * 128, 128)
                    cps.append(pltpu.async_copy(
                        tab_hbm.at[pl.ds(row0, 8), pl.ds(col, 128)],
                        out_hbm.at[tbase + tix], sem))
                for cp in cps:
                    cp.wait()
                return carry

            lax.fori_loop(0, _QT // _FIRE, chunk, 0)

        tbase_ent = kr * _TPG
        tbase_rel = (4 + kr) * _TPG

        @pl.when(wid < 16)
        def _():
            run(entT_hbm, tbase_ent)

        @pl.when(wid >= 16)
        def _():
            run(relT_hbm, tbase_rel)

        @pl.when(wid < 8)
        def _():
            g = wid
            pltpu.sync_copy(tail_hbm.at[g], out_hbm.at[g * _TPG + _TPG - 1])

    return detile


@functools.lru_cache(maxsize=None)
def _build_score(n: int):
    C = n // _NW
    V = C // _L
    mesh = plsc.VectorSubcoreMesh(core_axis_name="c", subcore_axis_name="s")

    @functools.partial(
        pl.kernel,
        out_type=(
            jax.ShapeDtypeStruct((n,), jnp.float32),
            jax.ShapeDtypeStruct((n * _ETA,), jnp.float32),
        ),
        mesh=mesh,
        compiler_params=pltpu.CompilerParams(
            needs_layout_passes=False, use_tc_tiling_on_sc=False),
        scratch_types=[
            pltpu.VMEM((C,), jnp.int32),
            pltpu.VMEM((C,), jnp.int32),
            pltpu.VMEM((C,), jnp.int32),
            pltpu.VMEM((_ETA * C,), jnp.int32),
            pltpu.VMEM((_ETA, C), jnp.int32),
            pltpu.VMEM((C,), jnp.int32),
            pltpu.VMEM((C,), jnp.int32),
            pltpu.VMEM((C,), jnp.int32),
            pltpu.VMEM((_ETA * C,), jnp.int32),
            pltpu.VMEM((2 * C,), jnp.float32),
            pltpu.VMEM((C,), jnp.float32),
            pltpu.VMEM((_ETA * C,), jnp.float32),
            pltpu.VMEM((C,), jnp.float32),
            pltpu.VMEM((_ETA, C), jnp.float32),
            pltpu.SemaphoreType.DMA,
            pltpu.SemaphoreType.DMA,
            pltpu.SemaphoreType.DMA,
            pltpu.SemaphoreType.DMA,
        ],
    )
    def score(tab_hbm, s_hbm, p_hbm, o_hbm, repl_hbm, sel_hbm,
              out_inp, out_corr,
              s_v, p_v, o_v, repl_v, sel_v, s_adj, p_adj, o_adj, repl_adj,
              eseo_v, ep_v, er_v, acc_inp, acc_corr,
              sem0, sem1, sem2, sem3):
        cid = lax.axis_index("c")
        sid = lax.axis_index("s")
        wid = sid * _NC + cid
        base_row = wid * C

        pltpu.sync_copy(s_hbm.at[wid], s_v)
        pltpu.sync_copy(p_hbm.at[wid], p_v)
        pltpu.sync_copy(o_hbm.at[wid], o_v)
        pltpu.sync_copy(repl_hbm.at[wid], repl_v)
        pltpu.sync_copy(sel_hbm.at[wid], sel_v)

        zeros = jnp.zeros((_L,), jnp.float32)
        for i in range(V):
            acc_inp[pl.ds(i * _L, _L)] = zeros
            for t in range(_ETA):
                acc_corr[t, pl.ds(i * _L, _L)] = zeros

        def body(k, carry):
            kadj = (k // 8) * _GSTRIDE + (k % 8) * 128
            kadj_rel = kadj + 4 * _GSTRIDE
            ka = jnp.full((_L,), 0, jnp.int32) + kadj
            kar = jnp.full((_L,), 0, jnp.int32) + kadj_rel
            for i in range(V):
                b = i * _L
                s_adj[pl.ds(b, _L)] = s_v[pl.ds(b, _L)] + ka
                o_adj[pl.ds(b, _L)] = o_v[pl.ds(b, _L)] + ka
                p_adj[pl.ds(b, _L)] = p_v[pl.ds(b, _L)] + kar
            for i in range(_ETA * V):
                b = i * _L
                repl_adj[pl.ds(b, _L)] = repl_v[pl.ds(b, _L)] + ka

            cp_es = pltpu.async_copy(
                tab_hbm.at[s_adj], eseo_v.at[pl.ds(0, C)], sem0)
            cp_eo = pltpu.async_copy(
                tab_hbm.at[o_adj], eseo_v.at[pl.ds(C, C)], sem1)
            cp_ep = pltpu.async_copy(tab_hbm.at[p_adj], ep_v, sem2)
            cp_er = pltpu.async_copy(tab_hbm.at[repl_adj], er_v, sem3)
            cp_es.wait()
            cp_eo.wait()
            cp_ep.wait()
            cp_er.wait()

            for i in range(V):
                b = i * _L
                es = eseo_v[pl.ds(b, _L)]
                eo = eseo_v[pl.ds(C + b, _L)]
                ep = ep_v[pl.ds(b, _L)]
                acc_inp[pl.ds(b, _L)] += es * ep * eo
                for t in range(_ETA):
                    sel16 = sel_v[t, pl.ds(b, _L)]
                    cv = plsc.load_gather(eseo_v, [sel16])
                    rv = er_v[pl.ds(t * C + b, _L)]
                    acc_corr[t, pl.ds(b, _L)] += cv * ep * rv
            return carry

        lax.fori_loop(0, _K, body, 0)

        pltpu.sync_copy(acc_inp, out_inp.at[pl.ds(base_row, C)])
        for t in range(_ETA):
            pltpu.sync_copy(acc_corr.at[t],
                            out_corr.at[pl.ds(t * n + base_row, C)])

    return score


def _flat_base(e):
    return (e // 128) * 1024 + (e % 128)


def kernel(inputs, ent_emb, rel_emb):
    n = inputs.shape[0]
    n_ent = ent_emb.shape[0]
    C = n // _NW

    km, kr = jax.random.split(jax.random.key(42))
    keep_subj = jax.random.randint(km, (n * _ETA,), 0, 2, dtype=jnp.int32)
    replacements = jax.random.randint(kr, (n * _ETA,), 0, n_ent,
                                      dtype=jnp.int32)
    keep_obj = 1 - keep_subj

    s = _flat_base(inputs[:, 0]).reshape(_NW, C)
    p = _flat_base(inputs[:, 1]).reshape(_NW, C)
    o = _flat_base(inputs[:, 2]).reshape(_NW, C)
    repl = _flat_base(replacements).reshape(_ETA, _NW, C).transpose(1, 0, 2)
    repl_flat = repl.reshape(_NW, _ETA * C)
    sel = (jnp.arange(C, dtype=jnp.int32)[None, None, :]
           + C * keep_obj.reshape(_ETA, _NW, C).transpose(1, 0, 2))

    tail = jnp.concatenate(
        [ent_emb.T[:, _ALIGNED:], rel_emb.T[:, _ALIGNED:]])
    tail = tail.reshape(8, 8, _N_ENT - _ALIGNED)
    tail = jnp.pad(tail, ((0, 0), (0, 0), (0, 128 - (_N_ENT - _ALIGNED))))

    tabL = _build_detile()(ent_emb.T, rel_emb.T, tail)
    tab_flat = tabL.reshape(-1)
    inp_score, corr_score = _build_score(n)(
        tab_flat, s, p, o, repl_flat, sel)
    return (inp_score, corr_score)


# speedup vs baseline: 13.7707x; 13.7707x over previous
import functools

import jax
import jax.numpy as jnp
from jax import lax
from jax.experimental import pallas as pl
from jax.experimental.pallas import tpu as pltpu
from jax.experimental.pallas import tpu_sc as plsc

_ETA = 10
_K = 32
_NC = 2
_NS = 16
_NW = _NC * _NS
_L = 16

_N_ENT = 1000000
_ALIGNED = 999936
_TPG = 7813
_GSTRIDE = _TPG * 1024
_NTILES = 8 * _TPG
_QT = 7812 // 4
_CT = 21
_CW = _CT * 128
_NCH = _QT // _CT


@functools.lru_cache(maxsize=None)
def _build_detile():
    mesh = plsc.VectorSubcoreMesh(core_axis_name="c", subcore_axis_name="s")

    @functools.partial(
        pl.kernel,
        out_type=jax.ShapeDtypeStruct((_NTILES * 8, 128), jnp.float32),
        mesh=mesh,
        compiler_params=pltpu.CompilerParams(
            needs_layout_passes=False, use_tc_tiling_on_sc=True),
        scratch_types=[
            pltpu.VMEM((8, _CW), jnp.float32),
            pltpu.VMEM((8, _CW), jnp.float32),
            pltpu.VMEM((_CT * 8, 128), jnp.float32),
            pltpu.VMEM((_CT * 8, 128), jnp.float32),
            pltpu.SemaphoreType.DMA,
            pltpu.SemaphoreType.DMA,
            pltpu.SemaphoreType.DMA,
            pltpu.SemaphoreType.DMA,
        ],
    )
    def detile(entT_hbm, relT_hbm, tail_hbm, out_hbm,
               bufT0, bufT1, buf20, buf21, semr0, semr1, semw0, semw1):
        cid = lax.axis_index("c")
        sid = lax.axis_index("s")
        wid = sid * _NC + cid
        rg = wid % 16
        kr = rg // 4
        q = rg % 4
        row0 = pl.multiple_of(8 * kr, 8)
        t0 = q * _QT

        bufTs = (bufT0, bufT1)
        buf2s = (buf20, buf21)
        semrs = (semr0, semr1)
        semws = (semw0, semw1)

        def run(tab_hbm, tbase):
            def src(j):
                col = pl.multiple_of((t0 + j * _CT) * 128, 128)
                return tab_hbm.at[pl.ds(row0, 8), pl.ds(col, _CW)]

            def one(j, par, prefetch):
                if prefetch:
                    pltpu.async_copy(src(j + 1), bufTs[1 - par],
                                     semrs[1 - par])
                pltpu.make_async_copy(src(j), bufTs[par], semrs[par]).wait()

                @pl.when(j >= 2)
                def _():
                    pltpu.make_async_copy(
                        buf2s[par],
                        out_hbm.at[pl.ds(0, _CT * 8)], semws[par]).wait()

                bufT = bufTs[par]
                buf2 = buf2s[par]

                def tile_remap(m, c2):
                    for r in range(8):
                        for c in range(8):
                            buf2[m * 8 + r, pl.ds(c * 16, 16)] = (
                                bufT[r, pl.ds(m * 128 + c * 16, 16)])
                    return c2

                lax.fori_loop(0, _CT, tile_remap, 0)

                dst_row = pl.multiple_of((tbase + t0 + j * _CT) * 8, 8)
                pltpu.async_copy(
                    buf2, out_hbm.at[pl.ds(dst_row, _CT * 8)], semws[par])

            pltpu.async_copy(src(0), bufT0, semr0)

            def pair(i, carry):
                one(2 * i, 0, True)
                one(2 * i + 1, 1, True)
                return carry

            lax.fori_loop(0, (_NCH - 1) // 2, pair, 0)
            one(_NCH - 1, 0, False)
            for par in range(2):
                pltpu.make_async_copy(
                    buf2s[par], out_hbm.at[pl.ds(0, _CT * 8)],
                    semws[par]).wait()

        tbase_ent = kr * _TPG
        tbase_rel = (4 + kr) * _TPG

        @pl.when(wid < 16)
        def _():
            run(entT_hbm, tbase_ent)

        @pl.when(wid >= 16)
        def _():
            run(relT_hbm, tbase_rel)

        @pl.when(wid < 8)
        def _():
            g = wid
            pltpu.sync_copy(
                tail_hbm.at[g],
                out_hbm.at[pl.ds((g * _TPG + _TPG - 1) * 8, 8)])

    return detile


@functools.lru_cache(maxsize=None)
def _build_score(n: int):
    C = n // _NW
    V = C // _L
    mesh = plsc.VectorSubcoreMesh(core_axis_name="c", subcore_axis_name="s")

    @functools.partial(
        pl.kernel,
        out_type=(
            jax.ShapeDtypeStruct((n,), jnp.float32),
            jax.ShapeDtypeStruct((n * _ETA,), jnp.float32),
        ),
        mesh=mesh,
        compiler_params=pltpu.CompilerParams(
            needs_layout_passes=False, use_tc_tiling_on_sc=False),
        scratch_types=[
            pltpu.VMEM((C,), jnp.int32),
            pltpu.VMEM((C,), jnp.int32),
            pltpu.VMEM((C,), jnp.int32),
            pltpu.VMEM((_ETA * C,), jnp.int32),
            pltpu.VMEM((_ETA, C), jnp.int32),
            pltpu.VMEM((C,), jnp.int32),
            pltpu.VMEM((C,), jnp.int32),
            pltpu.VMEM((C,), jnp.int32),
            pltpu.VMEM((_ETA * C,), jnp.int32),
            pltpu.VMEM((2 * C,), jnp.float32),
            pltpu.VMEM((C,), jnp.float32),
            pltpu.VMEM((_ETA * C,), jnp.float32),
            pltpu.VMEM((C,), jnp.float32),
            pltpu.VMEM((_ETA, C), jnp.float32),
            pltpu.SemaphoreType.DMA,
            pltpu.SemaphoreType.DMA,
            pltpu.SemaphoreType.DMA,
            pltpu.SemaphoreType.DMA,
        ],
    )
    def score(tab_hbm, s_hbm, p_hbm, o_hbm, repl_hbm, sel_hbm,
              out_inp, out_corr,
              s_v, p_v, o_v, repl_v, sel_v, s_adj, p_adj, o_adj, repl_adj,
              eseo_v, ep_v, er_v, acc_inp, acc_corr,
              sem0, sem1, sem2, sem3):
        cid = lax.axis_index("c")
        sid = lax.axis_index("s")
        wid = sid * _NC + cid
        base_row = wid * C

        pltpu.sync_copy(s_hbm.at[wid], s_v)
        pltpu.sync_copy(p_hbm.at[wid], p_v)
        pltpu.sync_copy(o_hbm.at[wid], o_v)
        pltpu.sync_copy(repl_hbm.at[wid], repl_v)
        pltpu.sync_copy(sel_hbm.at[wid], sel_v)

        zeros = jnp.zeros((_L,), jnp.float32)
        for i in range(V):
            acc_inp[pl.ds(i * _L, _L)] = zeros
            for t in range(_ETA):
                acc_corr[t, pl.ds(i * _L, _L)] = zeros

        def body(k, carry):
            kadj = (k // 8) * _GSTRIDE + (k % 8) * 128
            kadj_rel = kadj + 4 * _GSTRIDE
            ka = jnp.full((_L,), 0, jnp.int32) + kadj
            kar = jnp.full((_L,), 0, jnp.int32) + kadj_rel
            for i in range(V):
                b = i * _L
                s_adj[pl.ds(b, _L)] = s_v[pl.ds(b, _L)] + ka
                o_adj[pl.ds(b, _L)] = o_v[pl.ds(b, _L)] + ka
                p_adj[pl.ds(b, _L)] = p_v[pl.ds(b, _L)] + kar
            for i in range(_ETA * V):
                b = i * _L
                repl_adj[pl.ds(b, _L)] = repl_v[pl.ds(b, _L)] + ka

            cp_es = pltpu.async_copy(
                tab_hbm.at[s_adj], eseo_v.at[pl.ds(0, C)], sem0)
            cp_eo = pltpu.async_copy(
                tab_hbm.at[o_adj], eseo_v.at[pl.ds(C, C)], sem1)
            cp_ep = pltpu.async_copy(tab_hbm.at[p_adj], ep_v, sem2)
            cp_er = pltpu.async_copy(tab_hbm.at[repl_adj], er_v, sem3)
            cp_es.wait()
            cp_eo.wait()
            cp_ep.wait()
            cp_er.wait()

            for i in range(V):
                b = i * _L
                es = eseo_v[pl.ds(b, _L)]
                eo = eseo_v[pl.ds(C + b, _L)]
                ep = ep_v[pl.ds(b, _L)]
                acc_inp[pl.ds(b, _L)] += es * ep * eo
                for t in range(_ETA):
                    sel16 = sel_v[t, pl.ds(b, _L)]
                    cv = plsc.load_gather(eseo_v, [sel16])
                    rv = er_v[pl.ds(t * C + b, _L)]
                    acc_corr[t, pl.ds(b, _L)] += cv * ep * rv
            return carry

        lax.fori_loop(0, _K, body, 0)

        pltpu.sync_copy(acc_inp, out_inp.at[pl.ds(base_row, C)])
        for t in range(_ETA):
            pltpu.sync_copy(acc_corr.at[t],
                            out_corr.at[pl.ds(t * n + base_row, C)])

    return score


def _flat_base(e):
    return (e // 128) * 1024 + (e % 128)


def kernel(inputs, ent_emb, rel_emb):
    n = inputs.shape[0]
    n_ent = ent_emb.shape[0]
    C = n // _NW

    km, kr = jax.random.split(jax.random.key(42))
    keep_subj = jax.random.randint(km, (n * _ETA,), 0, 2, dtype=jnp.int32)
    replacements = jax.random.randint(kr, (n * _ETA,), 0, n_ent,
                                      dtype=jnp.int32)
    keep_obj = 1 - keep_subj

    s = _flat_base(inputs[:, 0]).reshape(_NW, C)
    p = _flat_base(inputs[:, 1]).reshape(_NW, C)
    o = _flat_base(inputs[:, 2]).reshape(_NW, C)
    repl = _flat_base(replacements).reshape(_ETA, _NW, C).transpose(1, 0, 2)
    repl_flat = repl.reshape(_NW, _ETA * C)
    sel = (jnp.arange(C, dtype=jnp.int32)[None, None, :]
           + C * keep_obj.reshape(_ETA, _NW, C).transpose(1, 0, 2))

    tail = jnp.concatenate(
        [ent_emb.T[:, _ALIGNED:], rel_emb.T[:, _ALIGNED:]])
    tail = tail.reshape(8, 8, _N_ENT - _ALIGNED)
    tail = jnp.pad(tail, ((0, 0), (0, 0), (0, 128 - (_N_ENT - _ALIGNED))))

    tabL = _build_detile()(ent_emb.T, rel_emb.T, tail)
    tab_flat = tabL.reshape(-1)
    inp_score, corr_score = _build_score(n)(
        tab_flat, s, p, o, repl_flat, sel)
    return (inp_score, corr_score)


# speedup vs baseline: 17.4857x; 1.2698x over previous
import functools

import jax
import jax.numpy as jnp
from jax import lax
from jax.experimental import pallas as pl
from jax.experimental.pallas import tpu as pltpu
from jax.experimental.pallas import tpu_sc as plsc

_ETA = 10
_K = 32
_NC = 2
_NS = 16
_NW = _NC * _NS
_L = 16

_N_ENT = 1000000
_ALIGNED = 999936
_TPG = 7813
_GSTRIDE = _TPG * 1024
_NTILES = 8 * _TPG
_QT = 7812 // 4
_CT = 21
_CW = _CT * 128
_NCH = _QT // _CT


@functools.lru_cache(maxsize=None)
def _build_detile():
    mesh = plsc.VectorSubcoreMesh(core_axis_name="c", subcore_axis_name="s")

    @functools.partial(
        pl.kernel,
        out_type=jax.ShapeDtypeStruct((_NTILES * 8, 128), jnp.float32),
        mesh=mesh,
        compiler_params=pltpu.CompilerParams(
            needs_layout_passes=False, use_tc_tiling_on_sc=True),
        scratch_types=[
            pltpu.VMEM((8, _CW), jnp.float32),
            pltpu.VMEM((8, _CW), jnp.float32),
            pltpu.VMEM((_CT * 8, 128), jnp.float32),
            pltpu.VMEM((_CT * 8, 128), jnp.float32),
            pltpu.SemaphoreType.DMA,
            pltpu.SemaphoreType.DMA,
            pltpu.SemaphoreType.DMA,
            pltpu.SemaphoreType.DMA,
        ],
    )
    def detile(entT_hbm, relT_hbm, tail_hbm, out_hbm,
               bufT0, bufT1, buf20, buf21, semr0, semr1, semw0, semw1):
        cid = lax.axis_index("c")
        sid = lax.axis_index("s")
        wid = sid * _NC + cid
        rg = wid % 16
        kr = rg // 4
        q = rg % 4
        row0 = pl.multiple_of(8 * kr, 8)
        t0 = q * _QT

        bufTs = (bufT0, bufT1)
        buf2s = (buf20, buf21)
        semrs = (semr0, semr1)
        semws = (semw0, semw1)

        def run(tab_hbm, tbase):
            def src(j):
                col = pl.multiple_of((t0 + j * _CT) * 128, 128)
                return tab_hbm.at[pl.ds(row0, 8), pl.ds(col, _CW)]

            def one(j, par, prefetch):
                if prefetch:
                    pltpu.async_copy(src(j + 1), bufTs[1 - par],
                                     semrs[1 - par])
                pltpu.make_async_copy(src(j), bufTs[par], semrs[par]).wait()

                @pl.when(j >= 2)
                def _():
                    pltpu.make_async_copy(
                        buf2s[par],
                        out_hbm.at[pl.ds(0, _CT * 8)], semws[par]).wait()

                bufT = bufTs[par]
                buf2 = buf2s[par]

                def tile_remap(m, c2):
                    for r in range(8):
                        for c in range(8):
                            buf2[m * 8 + r, pl.ds(c * 16, 16)] = (
                                bufT[r, pl.ds(m * 128 + c * 16, 16)])
                    return c2

                lax.fori_loop(0, _CT, tile_remap, 0)

                dst_row = pl.multiple_of((tbase + t0 + j * _CT) * 8, 8)
                pltpu.async_copy(
                    buf2, out_hbm.at[pl.ds(dst_row, _CT * 8)], semws[par])

            pltpu.async_copy(src(0), bufT0, semr0)

            def pair(i, carry):
                one(2 * i, 0, True)
                one(2 * i + 1, 1, True)
                return carry

            lax.fori_loop(0, (_NCH - 1) // 2, pair, 0)
            one(_NCH - 1, 0, False)
            for par in range(2):
                pltpu.make_async_copy(
                    buf2s[par], out_hbm.at[pl.ds(0, _CT * 8)],
                    semws[par]).wait()

        tbase_ent = kr * _TPG
        tbase_rel = (4 + kr) * _TPG

        @pl.when(wid < 16)
        def _():
            run(entT_hbm, tbase_ent)

        @pl.when(wid >= 16)
        def _():
            run(relT_hbm, tbase_rel)

        @pl.when(wid < 8)
        def _():
            g = wid
            pltpu.sync_copy(
                tail_hbm.at[g],
                out_hbm.at[pl.ds((g * _TPG + _TPG - 1) * 8, 8)])

    return detile


@functools.lru_cache(maxsize=None)
def _build_score(n: int):
    C = n // _NW
    V = C // _L
    mesh = plsc.VectorSubcoreMesh(core_axis_name="c", subcore_axis_name="s")

    @functools.partial(
        pl.kernel,
        out_type=(
            jax.ShapeDtypeStruct((n,), jnp.float32),
            jax.ShapeDtypeStruct((n * _ETA,), jnp.float32),
        ),
        mesh=mesh,
        compiler_params=pltpu.CompilerParams(
            needs_layout_passes=False, use_tc_tiling_on_sc=False),
        scratch_types=[
            pltpu.VMEM((C,), jnp.int32),
            pltpu.VMEM((C,), jnp.int32),
            pltpu.VMEM((C,), jnp.int32),
            pltpu.VMEM((_ETA * C,), jnp.int32),
            pltpu.VMEM((_ETA, C), jnp.int32),
            pltpu.VMEM((2, C), jnp.int32),
            pltpu.VMEM((2, C), jnp.int32),
            pltpu.VMEM((2, C), jnp.int32),
            pltpu.VMEM((2, _ETA * C), jnp.int32),
            pltpu.VMEM((2, 2 * C), jnp.float32),
            pltpu.VMEM((2, C), jnp.float32),
            pltpu.VMEM((2, _ETA * C), jnp.float32),
            pltpu.VMEM((C,), jnp.float32),
            pltpu.VMEM((_ETA, C), jnp.float32),
            pltpu.SemaphoreType.DMA,
            pltpu.SemaphoreType.DMA,
            pltpu.SemaphoreType.DMA,
            pltpu.SemaphoreType.DMA,
            pltpu.SemaphoreType.DMA,
            pltpu.SemaphoreType.DMA,
            pltpu.SemaphoreType.DMA,
            pltpu.SemaphoreType.DMA,
        ],
    )
    def score(tab_hbm, s_hbm, p_hbm, o_hbm, repl_hbm, sel_hbm,
              out_inp, out_corr,
              s_v, p_v, o_v, repl_v, sel_v, s_adj, p_adj, o_adj, repl_adj,
              eseo_v, ep_v, er_v, acc_inp, acc_corr,
              sem_es0, sem_eo0, sem_ep0, sem_er0,
              sem_es1, sem_eo1, sem_ep1, sem_er1):
        cid = lax.axis_index("c")
        sid = lax.axis_index("s")
        wid = sid * _NC + cid
        base_row = wid * C
        sems = ((sem_es0, sem_eo0, sem_ep0, sem_er0),
                (sem_es1, sem_eo1, sem_ep1, sem_er1))

        pltpu.sync_copy(s_hbm.at[wid], s_v)
        pltpu.sync_copy(p_hbm.at[wid], p_v)
        pltpu.sync_copy(o_hbm.at[wid], o_v)
        pltpu.sync_copy(repl_hbm.at[wid], repl_v)
        pltpu.sync_copy(sel_hbm.at[wid], sel_v)

        zeros = jnp.zeros((_L,), jnp.float32)
        for i in range(V):
            acc_inp[pl.ds(i * _L, _L)] = zeros
            for t in range(_ETA):
                acc_corr[t, pl.ds(i * _L, _L)] = zeros

        def stage(k, par):
            kadj = (k // 8) * _GSTRIDE + (k % 8) * 128
            ka = jnp.full((_L,), 0, jnp.int32) + kadj
            kar = ka + 4 * _GSTRIDE
            for i in range(V):
                b = i * _L
                s_adj[par, pl.ds(b, _L)] = s_v[pl.ds(b, _L)] + ka
                o_adj[par, pl.ds(b, _L)] = o_v[pl.ds(b, _L)] + ka
                p_adj[par, pl.ds(b, _L)] = p_v[pl.ds(b, _L)] + kar

            def adj_t(t, carry):
                for i in range(V):
                    b = t * C + i * _L
                    repl_adj[par, pl.ds(b, _L)] = (
                        repl_v[pl.ds(b, _L)] + ka)
                return carry

            lax.fori_loop(0, _ETA, adj_t, 0)
            pltpu.async_copy(tab_hbm.at[s_adj.at[par]],
                             eseo_v.at[par, pl.ds(0, C)], sems[par][0])
            pltpu.async_copy(tab_hbm.at[o_adj.at[par]],
                             eseo_v.at[par, pl.ds(C, C)], sems[par][1])
            pltpu.async_copy(tab_hbm.at[p_adj.at[par]],
                             ep_v.at[par], sems[par][2])
            pltpu.async_copy(tab_hbm.at[repl_adj.at[par]],
                             er_v.at[par], sems[par][3])

        def compute(k, par, prefetch):
            if prefetch:
                stage(k + 1, 1 - par)
            pltpu.make_async_copy(tab_hbm.at[s_adj.at[par]],
                                  eseo_v.at[par, pl.ds(0, C)],
                                  sems[par][0]).wait()
            pltpu.make_async_copy(tab_hbm.at[o_adj.at[par]],
                                  eseo_v.at[par, pl.ds(C, C)],
                                  sems[par][1]).wait()
            pltpu.make_async_copy(tab_hbm.at[p_adj.at[par]],
                                  ep_v.at[par], sems[par][2]).wait()
            pltpu.make_async_copy(tab_hbm.at[repl_adj.at[par]],
                                  er_v.at[par], sems[par][3]).wait()
            eseo = eseo_v.at[par]
            for i in range(V):
                b = i * _L
                es = eseo_v[par, pl.ds(b, _L)]
                eo = eseo_v[par, pl.ds(C + b, _L)]
                ep = ep_v[par, pl.ds(b, _L)]
                acc_inp[pl.ds(b, _L)] += es * ep * eo

            def corr_t(t, carry):
                for i in range(V):
                    b = i * _L
                    ep = ep_v[par, pl.ds(b, _L)]
                    sel16 = sel_v[t, pl.ds(b, _L)]
                    cv = plsc.load_gather(eseo, [sel16])
                    rv = er_v[par, pl.ds(t * C + b, _L)]
                    acc_corr[t, pl.ds(b, _L)] += cv * ep * rv
                return carry

            lax.fori_loop(0, _ETA, corr_t, 0)

        stage(0, 0)

        def pair(i, carry):
            kk = 2 * i
            compute(kk, 0, True)
            compute(kk + 1, 1, True)
            return carry

        lax.fori_loop(0, _K // 2 - 1, pair, 0)
        compute(_K - 2, 0, True)
        compute(_K - 1, 1, False)

        pltpu.sync_copy(acc_inp, out_inp.at[pl.ds(base_row, C)])
        for t in range(_ETA):
            pltpu.sync_copy(acc_corr.at[t],
                            out_corr.at[pl.ds(t * n + base_row, C)])

    return score


def _flat_base(e):
    return (e // 128) * 1024 + (e % 128)


def kernel(inputs, ent_emb, rel_emb):
    n = inputs.shape[0]
    n_ent = ent_emb.shape[0]
    C = n // _NW

    km, kr = jax.random.split(jax.random.key(42))
    keep_subj = jax.random.randint(km, (n * _ETA,), 0, 2, dtype=jnp.int32)
    replacements = jax.random.randint(kr, (n * _ETA,), 0, n_ent,
                                      dtype=jnp.int32)
    keep_obj = 1 - keep_subj

    s = _flat_base(inputs[:, 0]).reshape(_NW, C)
    p = _flat_base(inputs[:, 1]).reshape(_NW, C)
    o = _flat_base(inputs[:, 2]).reshape(_NW, C)
    repl = _flat_base(replacements).reshape(_ETA, _NW, C).transpose(1, 0, 2)
    repl_flat = repl.reshape(_NW, _ETA * C)
    sel = (jnp.arange(C, dtype=jnp.int32)[None, None, :]
           + C * keep_obj.reshape(_ETA, _NW, C).transpose(1, 0, 2))

    tail = jnp.concatenate(
        [ent_emb.T[:, _ALIGNED:], rel_emb.T[:, _ALIGNED:]])
    tail = tail.reshape(8, 8, _N_ENT - _ALIGNED)
    tail = jnp.pad(tail, ((0, 0), (0, 0), (0, 128 - (_N_ENT - _ALIGNED))))

    tabL = _build_detile()(ent_emb.T, rel_emb.T, tail)
    tab_flat = tabL.reshape(-1)
    inp_score, corr_score = _build_score(n)(
        tab_flat, s, p, o, repl_flat, sel)
    return (inp_score, corr_score)
